# baseline (speedup 1.0000x reference)
import functools

import jax
import jax.numpy as jnp
from jax import lax
from jax.experimental import pallas as pl
from jax.experimental.pallas import tpu as pltpu
from jax.experimental.pallas import tpu_sc as plsc

BLK = 4096
TCHUNK = 256
SC_CH = 128
SC_NW = 32
F32 = jnp.float32
I32 = jnp.int32


def _round_up(x: int, m: int) -> int:
    return (x + m - 1) // m * m


def _gather_sc(table, gidx, e_pad, d):
    bpw = e_pad // SC_NW
    nchunk = bpw // SC_CH
    mesh = plsc.VectorSubcoreMesh(core_axis_name="c", subcore_axis_name="s")

    @functools.partial(
        pl.kernel,
        out_type=jax.ShapeDtypeStruct((e_pad, d), F32),
        mesh=mesh,
        scratch_types=[
            pltpu.VMEM((bpw,), I32),
            pltpu.VMEM((3, SC_CH, d), F32),
            pltpu.SemaphoreType.DMA((3,)),
            pltpu.SemaphoreType.DMA((3,)),
        ],
    )
    def k(table_hbm, idx_hbm, out_hbm, idx_v, rows_v, gsem, wsem):
        wid = lax.axis_index("s") * 2 + lax.axis_index("c")
        base = wid * bpw
        pltpu.sync_copy(idx_hbm.at[pl.ds(base, bpw)], idx_v)

        def start_gather(j):
            sl = lax.rem(j, 3)
            pltpu.make_async_copy(
                table_hbm.at[idx_v.at[pl.ds(j * SC_CH, SC_CH)]],
                rows_v.at[sl], gsem.at[sl]).start()

        def wait_gather(j):
            sl = lax.rem(j, 3)
            pltpu.make_async_copy(
                table_hbm.at[idx_v.at[pl.ds(j * SC_CH, SC_CH)]],
                rows_v.at[sl], gsem.at[sl]).wait()

        def start_write(j):
            sl = lax.rem(j, 3)
            pltpu.make_async_copy(
                rows_v.at[sl], out_hbm.at[pl.ds(base + j * SC_CH, SC_CH)],
                wsem.at[sl]).start()

        def wait_write(j):
            sl = lax.rem(j, 3)
            pltpu.make_async_copy(
                rows_v.at[sl], out_hbm.at[pl.ds(base + j * SC_CH, SC_CH)],
                wsem.at[sl]).wait()

        start_gather(0)
        if nchunk > 1:
            start_gather(1)

        def body(j, carry):
            @pl.when(jnp.logical_and(j >= 1, j + 2 < nchunk))
            def _():
                wait_write(j - 1)

            @pl.when(j + 2 < nchunk)
            def _():
                start_gather(j + 2)

            wait_gather(j)
            start_write(j)
            return carry

        lax.fori_loop(0, nchunk, body, 0)
        for jj in range(max(0, nchunk - 3), nchunk):
            wait_write(jj)

    return k(table, gidx)


def _stats_kernel(feat_ref, gamma_ref, beta_ref, out_ref):
    f = feat_ref[:]
    mean = jnp.mean(f, axis=0, keepdims=True)
    var = jnp.mean((f - mean) * (f - mean), axis=0, keepdims=True)
    scale = gamma_ref[:] * lax.rsqrt(var + 1e-5)
    out_ref[0:1, :] = scale
    out_ref[1:2, :] = beta_ref[:] - mean * scale


def _bn_stats(feat, gamma, beta):
    n, d = feat.shape
    return pl.pallas_call(
        _stats_kernel,
        out_shape=jax.ShapeDtypeStruct((2, d), F32),
    )(feat, gamma.reshape(1, d), beta.reshape(1, d))


def _gru_kernel(total_ref, off_ref, r0_ref, act_ref, stats_ref, wih_ref,
                whh_ref, bi_ref, bh_ref, xg_ref, h_ref, out_ref, xbuf, sem,
                *, d, hdim):
    out_ref[:] = h_ref[:]
    total = total_ref[0]
    scale = stats_ref[0:1, :]
    shift = stats_ref[1:2, :]
    wih = wih_ref[:]
    whh = whh_ref[:]
    bi = bi_ref[:]
    bh = bh_ref[:]

    def issue(j):
        sl = lax.rem(j, 2)
        pltpu.make_async_copy(
            xg_ref.at[pl.ds(off_ref[j], BLK)], xbuf.at[sl], sem.at[sl]
        ).start()

    issue(0)

    def body(i, carry):
        sl = lax.rem(i, 2)

        @pl.when(i + 1 < total)
        def _():
            issue(i + 1)

        pltpu.make_async_copy(
            xg_ref.at[pl.ds(off_ref[i], BLK)], xbuf.at[sl], sem.at[sl]
        ).wait()
        r0 = r0_ref[i]
        act = act_ref[i]
        x = xbuf[sl] * scale + shift
        hblk = out_ref[pl.ds(r0, BLK), :]
        xp = jnp.dot(x, wih, preferred_element_type=F32) + bi
        hp = jnp.dot(hblk, whh, preferred_element_type=F32) + bh
        rr = jax.nn.sigmoid(xp[:, 0:hdim] + hp[:, 0:hdim])
        zz = jax.nn.sigmoid(xp[:, hdim:2 * hdim] + hp[:, hdim:2 * hdim])
        nn = jnp.tanh(xp[:, 2 * hdim:3 * hdim] + rr * hp[:, 2 * hdim:3 * hdim])
        hnew = (1.0 - zz) * nn + zz * hblk
        rows = lax.broadcasted_iota(I32, (BLK, 1), 0)
        out_ref[pl.ds(r0, BLK), :] = jnp.where(rows < act, hnew, hblk)
        return carry

    lax.fori_loop(0, total, body, 0)


def _gru_rounds(total, off_i, r0_i, act_i, stats, wih_t, whh_t, bi, bh, xg, h):
    npad, hdim = h.shape
    d = xg.shape[1]
    in_specs = [
            pl.BlockSpec(memory_space=pltpu.MemorySpace.SMEM),
            pl.BlockSpec(memory_space=pltpu.MemorySpace.SMEM),
            pl.BlockSpec(memory_space=pltpu.MemorySpace.SMEM),
            pl.BlockSpec(memory_space=pltpu.MemorySpace.SMEM),
            pl.BlockSpec(memory_space=pltpu.MemorySpace.VMEM),
            pl.BlockSpec(memory_space=pltpu.MemorySpace.VMEM),
            pl.BlockSpec(memory_space=pltpu.MemorySpace.VMEM),
            pl.BlockSpec(memory_space=pltpu.MemorySpace.VMEM),
            pl.BlockSpec(memory_space=pltpu.MemorySpace.VMEM),
            pl.BlockSpec(memory_space=pltpu.MemorySpace.HBM),
            pl.BlockSpec(memory_space=pltpu.MemorySpace.VMEM),
        ]
    return pl.pallas_call(
        functools.partial(_gru_kernel, d=d, hdim=hdim),
        in_specs=in_specs,
        out_specs=pl.BlockSpec(memory_space=pltpu.MemorySpace.VMEM),
        out_shape=jax.ShapeDtypeStruct((npad, hdim), F32),
        scratch_shapes=[
            pltpu.VMEM((2, BLK, d), F32),
            pltpu.SemaphoreType.DMA((2,)),
        ],
    )(total, off_i, r0_i, act_i, stats, wih_t, whh_t, bi, bh, xg, h)


def _out_kernel(feat_ref, stats_ref, hn_ref, ws_ref, wn_ref, out_ref):
    fb = feat_ref[:] * stats_ref[0:1, :] + stats_ref[1:2, :]
    out_ref[:] = (jnp.dot(fb, ws_ref[:], preferred_element_type=F32)
                  + jnp.dot(hn_ref[:], wn_ref[:], preferred_element_type=F32))


def _out_proj(feat, stats, hn, ws_t, wn_t):
    n, d = feat.shape
    o = ws_t.shape[1]
    rb = 2000
    grid = (n // rb,) if n % rb == 0 else None
    if grid is None:
        rb = n
        grid = (1,)
    return pl.pallas_call(
        _out_kernel,
        grid=grid,
        in_specs=[
            pl.BlockSpec((rb, d), lambda i: (i, 0)),
            pl.BlockSpec((2, d), lambda i: (0, 0)),
            pl.BlockSpec((rb, hn.shape[1]), lambda i: (i, 0)),
            pl.BlockSpec((d, o), lambda i: (0, 0)),
            pl.BlockSpec((hn.shape[1], o), lambda i: (0, 0)),
        ],
        out_specs=pl.BlockSpec((rb, o), lambda i: (i, 0)),
        out_shape=jax.ShapeDtypeStruct((n, o), F32),
    )(feat, stats, hn, ws_t, wn_t)


def kernel(feat, edge_index, bn_gamma, bn_beta, W_ih, W_hh, b_ih, b_hh,
           W_self, W_neigh):
    n, d = feat.shape
    e = edge_index.shape[1]
    hdim = W_hh.shape[1]
    src = edge_index[0].astype(I32)
    dst = edge_index[1].astype(I32)

    sdst, ssrc = lax.sort((dst, src), num_keys=1)
    idx = jnp.arange(e, dtype=I32)
    new_run = jnp.concatenate([jnp.ones((1,), bool), sdst[1:] != sdst[:-1]])
    is_last = jnp.concatenate([sdst[1:] != sdst[:-1], jnp.ones((1,), bool)])
    run_start = lax.cummax(jnp.where(new_run, idx, 0))
    run_end = lax.cummin(jnp.where(is_last, idx + 1, e), axis=0, reverse=True)
    t_within = idx - run_start
    neg_deg = run_start - run_end
    t_sorted, _, xsrc = lax.sort((t_within, neg_deg, ssrc), num_keys=2)
    max_deg = t_sorted[e - 1] + 1
    deg = jnp.zeros((n,), I32).at[dst].add(1)
    nodeorder = jnp.argsort(-deg)
    rank = jnp.zeros((n,), I32).at[nodeorder].set(jnp.arange(n, dtype=I32))
    e_pad = _round_up(e + BLK, SC_NW * SC_CH)
    gidx = jnp.concatenate([xsrc, jnp.zeros((e_pad - e,), I32)])

    stats = _bn_stats(feat, bn_gamma, bn_beta)
    xg = _gather_sc(feat, gidx, e_pad, d)

    wih_t = W_ih.T.astype(F32)
    whh_t = W_hh.T.astype(F32)
    bi = b_ih.reshape(1, -1).astype(F32)
    bh = b_hh.reshape(1, -1).astype(F32)

    npad = _round_up(n, BLK)
    bmax = (e + BLK - 1) // BLK + TCHUNK
    h0 = jnp.zeros((npad, hdim), F32)

    def cond(state):
        t0, _ = state
        return t0 < max_deg

    def rbody(state):
        t0, h = state
        tq = t0 + jnp.arange(TCHUNK, dtype=I32)
        off_c = jnp.searchsorted(t_sorted, tq, side="left").astype(I32)
        off_r = jnp.searchsorted(t_sorted, tq, side="right").astype(I32)
        cnt_c = off_r - off_c
        nblk_t = (cnt_c + BLK - 1) // BLK
        bend = jnp.cumsum(nblk_t)
        total = bend[TCHUNK - 1]
        i_arr = jnp.arange(bmax, dtype=I32)
        t_i = jnp.minimum(jnp.searchsorted(bend, i_arr, side="right"),
                          TCHUNK - 1).astype(I32)
        b_i = i_arr - (bend[t_i] - nblk_t[t_i])
        r0_i = b_i * BLK
        off_i = off_c[t_i] + r0_i
        act_i = jnp.where(i_arr < total, cnt_c[t_i] - r0_i, 0)
        h = _gru_rounds(total.reshape(1), off_i, r0_i, act_i, stats,
                        wih_t, whh_t, bi, bh, xg, h)
        return (t0 + TCHUNK, h)

    _, h = lax.while_loop(cond, rbody, (jnp.int32(0), h0))

    n_pad = _round_up(n, SC_NW * SC_CH)
    rank_pad = jnp.concatenate([rank, jnp.zeros((n_pad - n,), I32)])
    hn = _gather_sc(h, rank_pad, n_pad, hdim)[:n]
    rst = _out_proj(feat, stats, hn, W_self.T.astype(F32),
                    W_neigh.T.astype(F32))
    return rst


# speedup vs baseline: 1.2284x; 1.2284x over previous
import functools

import jax
import jax.numpy as jnp
from jax import lax
from jax.experimental import pallas as pl
from jax.experimental.pallas import tpu as pltpu
from jax.experimental.pallas import tpu_sc as plsc

BLK = 2048
TCHUNK = 256
SC_CH = 128
SC_NW = 32
F32 = jnp.float32
I32 = jnp.int32


def _round_up(x: int, m: int) -> int:
    return (x + m - 1) // m * m


def _gather_sc(table, gidx, e_pad, d):
    bpw = e_pad // SC_NW
    nchunk = bpw // SC_CH
    mesh = plsc.VectorSubcoreMesh(core_axis_name="c", subcore_axis_name="s")

    @functools.partial(
        pl.kernel,
        out_type=jax.ShapeDtypeStruct((e_pad, d), F32),
        mesh=mesh,
        scratch_types=[
            pltpu.VMEM((bpw,), I32),
            pltpu.VMEM((3, SC_CH, d), F32),
            pltpu.SemaphoreType.DMA((3,)),
            pltpu.SemaphoreType.DMA((3,)),
        ],
    )
    def k(table_hbm, idx_hbm, out_hbm, idx_v, rows_v, gsem, wsem):
        wid = lax.axis_index("s") * 2 + lax.axis_index("c")
        base = wid * bpw
        pltpu.sync_copy(idx_hbm.at[pl.ds(base, bpw)], idx_v)

        def start_gather(j):
            sl = lax.rem(j, 3)
            pltpu.make_async_copy(
                table_hbm.at[idx_v.at[pl.ds(j * SC_CH, SC_CH)]],
                rows_v.at[sl], gsem.at[sl]).start()

        def wait_gather(j):
            sl = lax.rem(j, 3)
            pltpu.make_async_copy(
                table_hbm.at[idx_v.at[pl.ds(j * SC_CH, SC_CH)]],
                rows_v.at[sl], gsem.at[sl]).wait()

        def start_write(j):
            sl = lax.rem(j, 3)
            pltpu.make_async_copy(
                rows_v.at[sl], out_hbm.at[pl.ds(base + j * SC_CH, SC_CH)],
                wsem.at[sl]).start()

        def wait_write(j):
            sl = lax.rem(j, 3)
            pltpu.make_async_copy(
                rows_v.at[sl], out_hbm.at[pl.ds(base + j * SC_CH, SC_CH)],
                wsem.at[sl]).wait()

        start_gather(0)
        if nchunk > 1:
            start_gather(1)

        def body(j, carry):
            @pl.when(jnp.logical_and(j >= 1, j + 2 < nchunk))
            def _():
                wait_write(j - 1)

            @pl.when(j + 2 < nchunk)
            def _():
                start_gather(j + 2)

            wait_gather(j)
            start_write(j)
            return carry

        lax.fori_loop(0, nchunk, body, 0)
        for jj in range(max(0, nchunk - 3), nchunk):
            wait_write(jj)

    return k(table, gidx)


def _stats_kernel(feat_ref, gamma_ref, beta_ref, out_ref):
    f = feat_ref[:]
    mean = jnp.mean(f, axis=0, keepdims=True)
    var = jnp.mean((f - mean) * (f - mean), axis=0, keepdims=True)
    scale = gamma_ref[:] * lax.rsqrt(var + 1e-5)
    out_ref[0:1, :] = scale
    out_ref[1:2, :] = beta_ref[:] - mean * scale


def _bn_stats(feat, gamma, beta):
    n, d = feat.shape
    return pl.pallas_call(
        _stats_kernel,
        out_shape=jax.ShapeDtypeStruct((2, d), F32),
    )(feat, gamma.reshape(1, d), beta.reshape(1, d))


def _gru_kernel(total_ref, off_ref, r0_ref, act_ref, stats_ref, wih_ref,
                whh_ref, bi_ref, bh_ref, xg_ref, h_ref, out_ref, xbuf, sem,
                *, d, hdim):
    out_ref[:] = h_ref[:]
    total = total_ref[0]
    scale = stats_ref[0:1, :]
    shift = stats_ref[1:2, :]
    wih = wih_ref[:]
    whh = whh_ref[:]
    bi = bi_ref[:]
    bh = bh_ref[:]

    def issue(j):
        sl = lax.rem(j, 2)
        pltpu.make_async_copy(
            xg_ref.at[pl.ds(off_ref[j], BLK)], xbuf.at[sl], sem.at[sl]
        ).start()

    issue(0)

    def body(i, carry):
        sl = lax.rem(i, 2)

        @pl.when(i + 1 < total)
        def _():
            issue(i + 1)

        pltpu.make_async_copy(
            xg_ref.at[pl.ds(off_ref[i], BLK)], xbuf.at[sl], sem.at[sl]
        ).wait()
        r0 = r0_ref[i]
        act = act_ref[i]
        x = xbuf[sl] * scale + shift
        hblk = out_ref[pl.ds(r0, BLK), :]
        xp = jnp.dot(x, wih, preferred_element_type=F32) + bi
        hp = jnp.dot(hblk, whh, preferred_element_type=F32) + bh
        rr = jax.nn.sigmoid(xp[:, 0:hdim] + hp[:, 0:hdim])
        zz = jax.nn.sigmoid(xp[:, hdim:2 * hdim] + hp[:, hdim:2 * hdim])
        nn = jnp.tanh(xp[:, 2 * hdim:3 * hdim] + rr * hp[:, 2 * hdim:3 * hdim])
        hnew = (1.0 - zz) * nn + zz * hblk
        rows = lax.broadcasted_iota(I32, (BLK, 1), 0)
        out_ref[pl.ds(r0, BLK), :] = jnp.where(rows < act, hnew, hblk)
        return carry

    lax.fori_loop(0, total, body, 0)


def _gru_rounds(total, off_i, r0_i, act_i, stats, wih_t, whh_t, bi, bh, xg, h):
    npad, hdim = h.shape
    d = xg.shape[1]
    in_specs = [
            pl.BlockSpec(memory_space=pltpu.MemorySpace.SMEM),
            pl.BlockSpec(memory_space=pltpu.MemorySpace.SMEM),
            pl.BlockSpec(memory_space=pltpu.MemorySpace.SMEM),
            pl.BlockSpec(memory_space=pltpu.MemorySpace.SMEM),
            pl.BlockSpec(memory_space=pltpu.MemorySpace.VMEM),
            pl.BlockSpec(memory_space=pltpu.MemorySpace.VMEM),
            pl.BlockSpec(memory_space=pltpu.MemorySpace.VMEM),
            pl.BlockSpec(memory_space=pltpu.MemorySpace.VMEM),
            pl.BlockSpec(memory_space=pltpu.MemorySpace.VMEM),
            pl.BlockSpec(memory_space=pltpu.MemorySpace.HBM),
            pl.BlockSpec(memory_space=pltpu.MemorySpace.VMEM),
        ]
    return pl.pallas_call(
        functools.partial(_gru_kernel, d=d, hdim=hdim),
        in_specs=in_specs,
        out_specs=pl.BlockSpec(memory_space=pltpu.MemorySpace.VMEM),
        out_shape=jax.ShapeDtypeStruct((npad, hdim), F32),
        scratch_shapes=[
            pltpu.VMEM((2, BLK, d), F32),
            pltpu.SemaphoreType.DMA((2,)),
        ],
    )(total, off_i, r0_i, act_i, stats, wih_t, whh_t, bi, bh, xg, h)


def _out_kernel(feat_ref, stats_ref, hn_ref, ws_ref, wn_ref, out_ref):
    fb = feat_ref[:] * stats_ref[0:1, :] + stats_ref[1:2, :]
    out_ref[:] = (jnp.dot(fb, ws_ref[:], preferred_element_type=F32)
                  + jnp.dot(hn_ref[:], wn_ref[:], preferred_element_type=F32))


def _out_proj(feat, stats, hn, ws_t, wn_t):
    n, d = feat.shape
    o = ws_t.shape[1]
    rb = 2000
    grid = (n // rb,) if n % rb == 0 else None
    if grid is None:
        rb = n
        grid = (1,)
    return pl.pallas_call(
        _out_kernel,
        grid=grid,
        in_specs=[
            pl.BlockSpec((rb, d), lambda i: (i, 0)),
            pl.BlockSpec((2, d), lambda i: (0, 0)),
            pl.BlockSpec((rb, hn.shape[1]), lambda i: (i, 0)),
            pl.BlockSpec((d, o), lambda i: (0, 0)),
            pl.BlockSpec((hn.shape[1], o), lambda i: (0, 0)),
        ],
        out_specs=pl.BlockSpec((rb, o), lambda i: (i, 0)),
        out_shape=jax.ShapeDtypeStruct((n, o), F32),
    )(feat, stats, hn, ws_t, wn_t)


def kernel(feat, edge_index, bn_gamma, bn_beta, W_ih, W_hh, b_ih, b_hh,
           W_self, W_neigh):
    n, d = feat.shape
    e = edge_index.shape[1]
    hdim = W_hh.shape[1]
    src = edge_index[0].astype(I32)
    dst = edge_index[1].astype(I32)

    sdst, ssrc = lax.sort((dst, src), num_keys=1)
    idx = jnp.arange(e, dtype=I32)
    new_run = jnp.concatenate([jnp.ones((1,), bool), sdst[1:] != sdst[:-1]])
    is_last = jnp.concatenate([sdst[1:] != sdst[:-1], jnp.ones((1,), bool)])
    run_start = lax.cummax(jnp.where(new_run, idx, 0))
    run_end = lax.cummin(jnp.where(is_last, idx + 1, e), axis=0, reverse=True)
    t_within = idx - run_start
    neg_deg = run_start - run_end
    deg = jnp.zeros((n,), I32).at[dst].add(1)
    max_deg = jnp.max(deg)

    def _sort_packed(_):
        key = t_within * 8192 + (8191 + neg_deg)
        ks, xs = lax.sort((key, ssrc), num_keys=1)
        return ks // 8192, xs

    def _sort_general(_):
        ts, _, xs = lax.sort((t_within, neg_deg, ssrc), num_keys=2)
        return ts, xs

    t_sorted, xsrc = lax.cond(max_deg < 8192, _sort_packed, _sort_general,
                              None)
    nodeorder = jnp.argsort(-deg)
    rank = jnp.zeros((n,), I32).at[nodeorder].set(jnp.arange(n, dtype=I32))
    e_pad = _round_up(e + BLK, SC_NW * SC_CH)
    gidx = jnp.concatenate([xsrc, jnp.zeros((e_pad - e,), I32)])

    stats = _bn_stats(feat, bn_gamma, bn_beta)
    xg = _gather_sc(feat, gidx, e_pad, d)

    wih_t = W_ih.T.astype(F32)
    whh_t = W_hh.T.astype(F32)
    bi = b_ih.reshape(1, -1).astype(F32)
    bh = b_hh.reshape(1, -1).astype(F32)

    npad = _round_up(n, BLK)
    bmax = (e + BLK - 1) // BLK + TCHUNK
    h0 = jnp.zeros((npad, hdim), F32)

    def cond(state):
        t0, _ = state
        return t0 < max_deg

    def rbody(state):
        t0, h = state
        tq = t0 + jnp.arange(TCHUNK, dtype=I32)
        off_c = jnp.searchsorted(t_sorted, tq, side="left").astype(I32)
        off_r = jnp.searchsorted(t_sorted, tq, side="right").astype(I32)
        cnt_c = off_r - off_c
        nblk_t = (cnt_c + BLK - 1) // BLK
        bend = jnp.cumsum(nblk_t)
        total = bend[TCHUNK - 1]
        i_arr = jnp.arange(bmax, dtype=I32)
        t_i = jnp.minimum(jnp.searchsorted(bend, i_arr, side="right"),
                          TCHUNK - 1).astype(I32)
        b_i = i_arr - (bend[t_i] - nblk_t[t_i])
        r0_i = b_i * BLK
        off_i = off_c[t_i] + r0_i
        act_i = jnp.where(i_arr < total, cnt_c[t_i] - r0_i, 0)
        h = _gru_rounds(total.reshape(1), off_i, r0_i, act_i, stats,
                        wih_t, whh_t, bi, bh, xg, h)
        return (t0 + TCHUNK, h)

    _, h = lax.while_loop(cond, rbody, (jnp.int32(0), h0))

    n_pad = _round_up(n, SC_NW * SC_CH)
    rank_pad = jnp.concatenate([rank, jnp.zeros((n_pad - n,), I32)])
    hn = _gather_sc(h, rank_pad, n_pad, hdim)[:n]
    rst = _out_proj(feat, stats, hn, W_self.T.astype(F32),
                    W_neigh.T.astype(F32))
    return rst


# speedup vs baseline: 1.2297x; 1.0010x over previous
import functools

import jax
import jax.numpy as jnp
from jax import lax
from jax.experimental import pallas as pl
from jax.experimental.pallas import tpu as pltpu
from jax.experimental.pallas import tpu_sc as plsc

BLK = 2048
TCHUNK = 256
SC_CH = 128
SC_NW = 32
F32 = jnp.float32
I32 = jnp.int32


def _round_up(x: int, m: int) -> int:
    return (x + m - 1) // m * m


def _gather_sc(table, gidx, e_pad, d):
    bpw = e_pad // SC_NW
    nchunk = bpw // SC_CH
    mesh = plsc.VectorSubcoreMesh(core_axis_name="c", subcore_axis_name="s")

    @functools.partial(
        pl.kernel,
        out_type=jax.ShapeDtypeStruct((e_pad, d), F32),
        mesh=mesh,
        scratch_types=[
            pltpu.VMEM((bpw,), I32),
            pltpu.VMEM((4, SC_CH, d), F32),
            pltpu.SemaphoreType.DMA((4,)),
            pltpu.SemaphoreType.DMA((4,)),
        ],
    )
    def k(table_hbm, idx_hbm, out_hbm, idx_v, rows_v, gsem, wsem):
        wid = lax.axis_index("s") * 2 + lax.axis_index("c")
        base = wid * bpw
        pltpu.sync_copy(idx_hbm.at[pl.ds(base, bpw)], idx_v)

        def start_gather(j):
            sl = lax.rem(j, 4)
            pltpu.make_async_copy(
                table_hbm.at[idx_v.at[pl.ds(j * SC_CH, SC_CH)]],
                rows_v.at[sl], gsem.at[sl]).start()

        def wait_gather(j):
            sl = lax.rem(j, 4)
            pltpu.make_async_copy(
                table_hbm.at[idx_v.at[pl.ds(j * SC_CH, SC_CH)]],
                rows_v.at[sl], gsem.at[sl]).wait()

        def start_write(j):
            sl = lax.rem(j, 4)
            pltpu.make_async_copy(
                rows_v.at[sl], out_hbm.at[pl.ds(base + j * SC_CH, SC_CH)],
                wsem.at[sl]).start()

        def wait_write(j):
            sl = lax.rem(j, 4)
            pltpu.make_async_copy(
                rows_v.at[sl], out_hbm.at[pl.ds(base + j * SC_CH, SC_CH)],
                wsem.at[sl]).wait()

        for jj in range(min(3, nchunk)):
            start_gather(jj)

        def body(j, carry):
            @pl.when(jnp.logical_and(j >= 1, j + 3 < nchunk))
            def _():
                wait_write(j - 1)

            @pl.when(j + 3 < nchunk)
            def _():
                start_gather(j + 3)

            wait_gather(j)
            start_write(j)
            return carry

        lax.fori_loop(0, nchunk, body, 0)
        for jj in range(max(0, nchunk - 4), nchunk):
            wait_write(jj)

    return k(table, gidx)


def _stats_kernel(feat_ref, gamma_ref, beta_ref, out_ref):
    f = feat_ref[:]
    mean = jnp.mean(f, axis=0, keepdims=True)
    var = jnp.mean((f - mean) * (f - mean), axis=0, keepdims=True)
    scale = gamma_ref[:] * lax.rsqrt(var + 1e-5)
    out_ref[0:1, :] = scale
    out_ref[1:2, :] = beta_ref[:] - mean * scale


def _bn_stats(feat, gamma, beta):
    n, d = feat.shape
    return pl.pallas_call(
        _stats_kernel,
        out_shape=jax.ShapeDtypeStruct((2, d), F32),
    )(feat, gamma.reshape(1, d), beta.reshape(1, d))


def _gru_kernel(total_ref, off_ref, r0_ref, act_ref, stats_ref, wih_ref,
                whh_ref, bi_ref, bh_ref, xg_ref, h_ref, out_ref, xbuf, sem,
                *, d, hdim):
    out_ref[:] = h_ref[:]
    total = total_ref[0]
    scale = stats_ref[0:1, :]
    shift = stats_ref[1:2, :]
    wih = wih_ref[:]
    whh = whh_ref[:]
    bi = bi_ref[:]
    bh = bh_ref[:]

    def issue(j):
        sl = lax.rem(j, 2)
        pltpu.make_async_copy(
            xg_ref.at[pl.ds(off_ref[j], BLK)], xbuf.at[sl], sem.at[sl]
        ).start()

    issue(0)

    def body(i, carry):
        sl = lax.rem(i, 2)

        @pl.when(i + 1 < total)
        def _():
            issue(i + 1)

        pltpu.make_async_copy(
            xg_ref.at[pl.ds(off_ref[i], BLK)], xbuf.at[sl], sem.at[sl]
        ).wait()
        r0 = r0_ref[i]
        act = act_ref[i]
        x = xbuf[sl] * scale + shift
        hblk = out_ref[pl.ds(r0, BLK), :]
        xp = jnp.dot(x, wih, preferred_element_type=F32) + bi
        hp = jnp.dot(hblk, whh, preferred_element_type=F32) + bh
        rr = jax.nn.sigmoid(xp[:, 0:hdim] + hp[:, 0:hdim])
        zz = jax.nn.sigmoid(xp[:, hdim:2 * hdim] + hp[:, hdim:2 * hdim])
        nn = jnp.tanh(xp[:, 2 * hdim:3 * hdim] + rr * hp[:, 2 * hdim:3 * hdim])
        hnew = (1.0 - zz) * nn + zz * hblk
        rows = lax.broadcasted_iota(I32, (BLK, 1), 0)
        out_ref[pl.ds(r0, BLK), :] = jnp.where(rows < act, hnew, hblk)
        return carry

    lax.fori_loop(0, total, body, 0)


def _gru_rounds(total, off_i, r0_i, act_i, stats, wih_t, whh_t, bi, bh, xg, h):
    npad, hdim = h.shape
    d = xg.shape[1]
    in_specs = [
            pl.BlockSpec(memory_space=pltpu.MemorySpace.SMEM),
            pl.BlockSpec(memory_space=pltpu.MemorySpace.SMEM),
            pl.BlockSpec(memory_space=pltpu.MemorySpace.SMEM),
            pl.BlockSpec(memory_space=pltpu.MemorySpace.SMEM),
            pl.BlockSpec(memory_space=pltpu.MemorySpace.VMEM),
            pl.BlockSpec(memory_space=pltpu.MemorySpace.VMEM),
            pl.BlockSpec(memory_space=pltpu.MemorySpace.VMEM),
            pl.BlockSpec(memory_space=pltpu.MemorySpace.VMEM),
            pl.BlockSpec(memory_space=pltpu.MemorySpace.VMEM),
            pl.BlockSpec(memory_space=pltpu.MemorySpace.HBM),
            pl.BlockSpec(memory_space=pltpu.MemorySpace.VMEM),
        ]
    return pl.pallas_call(
        functools.partial(_gru_kernel, d=d, hdim=hdim),
        in_specs=in_specs,
        out_specs=pl.BlockSpec(memory_space=pltpu.MemorySpace.VMEM),
        out_shape=jax.ShapeDtypeStruct((npad, hdim), F32),
        scratch_shapes=[
            pltpu.VMEM((2, BLK, d), F32),
            pltpu.SemaphoreType.DMA((2,)),
        ],
    )(total, off_i, r0_i, act_i, stats, wih_t, whh_t, bi, bh, xg, h)


def _out_kernel(feat_ref, stats_ref, hn_ref, ws_ref, wn_ref, out_ref):
    fb = feat_ref[:] * stats_ref[0:1, :] + stats_ref[1:2, :]
    out_ref[:] = (jnp.dot(fb, ws_ref[:], preferred_element_type=F32)
                  + jnp.dot(hn_ref[:], wn_ref[:], preferred_element_type=F32))


def _out_proj(feat, stats, hn, ws_t, wn_t):
    n, d = feat.shape
    o = ws_t.shape[1]
    rb = 2000
    grid = (n // rb,) if n % rb == 0 else None
    if grid is None:
        rb = n
        grid = (1,)
    return pl.pallas_call(
        _out_kernel,
        grid=grid,
        in_specs=[
            pl.BlockSpec((rb, d), lambda i: (i, 0)),
            pl.BlockSpec((2, d), lambda i: (0, 0)),
            pl.BlockSpec((rb, hn.shape[1]), lambda i: (i, 0)),
            pl.BlockSpec((d, o), lambda i: (0, 0)),
            pl.BlockSpec((hn.shape[1], o), lambda i: (0, 0)),
        ],
        out_specs=pl.BlockSpec((rb, o), lambda i: (i, 0)),
        out_shape=jax.ShapeDtypeStruct((n, o), F32),
    )(feat, stats, hn, ws_t, wn_t)


def kernel(feat, edge_index, bn_gamma, bn_beta, W_ih, W_hh, b_ih, b_hh,
           W_self, W_neigh):
    n, d = feat.shape
    e = edge_index.shape[1]
    hdim = W_hh.shape[1]
    src = edge_index[0].astype(I32)
    dst = edge_index[1].astype(I32)

    sdst, ssrc = lax.sort((dst, src), num_keys=1)
    idx = jnp.arange(e, dtype=I32)
    new_run = jnp.concatenate([jnp.ones((1,), bool), sdst[1:] != sdst[:-1]])
    is_last = jnp.concatenate([sdst[1:] != sdst[:-1], jnp.ones((1,), bool)])
    run_start = lax.cummax(jnp.where(new_run, idx, 0))
    run_end = lax.cummin(jnp.where(is_last, idx + 1, e), axis=0, reverse=True)
    t_within = idx - run_start
    neg_deg = run_start - run_end
    deg = jnp.zeros((n,), I32).at[dst].add(1)
    max_deg = jnp.max(deg)

    def _sort_packed(_):
        key = t_within * 8192 + (8191 + neg_deg)
        ks, xs = lax.sort((key, ssrc), num_keys=1)
        return ks // 8192, xs

    def _sort_general(_):
        ts, _, xs = lax.sort((t_within, neg_deg, ssrc), num_keys=2)
        return ts, xs

    t_sorted, xsrc = lax.cond(max_deg < 8192, _sort_packed, _sort_general,
                              None)
    nodeorder = jnp.argsort(-deg)
    rank = jnp.zeros((n,), I32).at[nodeorder].set(jnp.arange(n, dtype=I32))
    e_pad = _round_up(e + BLK, SC_NW * SC_CH)
    gidx = jnp.concatenate([xsrc, jnp.zeros((e_pad - e,), I32)])

    stats = _bn_stats(feat, bn_gamma, bn_beta)
    xg = _gather_sc(feat, gidx, e_pad, d)

    wih_t = W_ih.T.astype(F32)
    whh_t = W_hh.T.astype(F32)
    bi = b_ih.reshape(1, -1).astype(F32)
    bh = b_hh.reshape(1, -1).astype(F32)

    npad = _round_up(n, BLK)
    bmax = (e + BLK - 1) // BLK + TCHUNK
    h0 = jnp.zeros((npad, hdim), F32)

    def cond(state):
        t0, _ = state
        return t0 < max_deg

    def rbody(state):
        t0, h = state
        tq = t0 + jnp.arange(TCHUNK, dtype=I32)
        off_c = jnp.searchsorted(t_sorted, tq, side="left").astype(I32)
        off_r = jnp.searchsorted(t_sorted, tq, side="right").astype(I32)
        cnt_c = off_r - off_c
        nblk_t = (cnt_c + BLK - 1) // BLK
        bend = jnp.cumsum(nblk_t)
        total = bend[TCHUNK - 1]
        i_arr = jnp.arange(bmax, dtype=I32)
        t_i = jnp.minimum(jnp.searchsorted(bend, i_arr, side="right"),
                          TCHUNK - 1).astype(I32)
        b_i = i_arr - (bend[t_i] - nblk_t[t_i])
        r0_i = b_i * BLK
        off_i = off_c[t_i] + r0_i
        act_i = jnp.where(i_arr < total, cnt_c[t_i] - r0_i, 0)
        h = _gru_rounds(total.reshape(1), off_i, r0_i, act_i, stats,
                        wih_t, whh_t, bi, bh, xg, h)
        return (t0 + TCHUNK, h)

    _, h = lax.while_loop(cond, rbody, (jnp.int32(0), h0))

    n_pad = _round_up(n, SC_NW * SC_CH)
    rank_pad = jnp.concatenate([rank, jnp.zeros((n_pad - n,), I32)])
    hn = _gather_sc(h, rank_pad, n_pad, hdim)[:n]
    rst = _out_proj(feat, stats, hn, W_self.T.astype(F32),
                    W_neigh.T.astype(F32))
    return rst
